# baseline (speedup 1.0000x reference)
import functools
import math

import jax
import jax.numpy as jnp
from jax import lax
from jax.experimental import pallas as pl
from jax.experimental.pallas import tpu as pltpu
from jax.experimental.pallas import tpu_sc as plsc

D = 64
SCALE = math.sqrt(D)
B = 4096
S = 200
NC = 2
NS = 16
NW = NC * NS
C = 128
TP = C + 1
NBUF = 4
NCHUNK = S
NGROUP = NCHUNK // NBUF


def _build_sc_embed():
  mesh = plsc.VectorSubcoreMesh(core_axis_name="c", subcore_axis_name="s")

  @functools.partial(
      pl.kernel,
      mesh=mesh,
      out_type=jax.ShapeDtypeStruct((S, D // 8, B // C, 8, C), jnp.float32),
      compiler_params=pltpu.CompilerParams(use_tc_tiling_on_sc=False,
                                           needs_layout_passes=False),
      scratch_types=[
          pltpu.VMEM((S // 8, 8, C), jnp.int32),
          pltpu.VMEM((NBUF, C, D), jnp.float32),
          pltpu.VMEM((NBUF, D // 8, 8, TP), jnp.float32),
          pltpu.SemaphoreType.DMA,
          pltpu.SemaphoreType.DMA((NBUF,)),
          pltpu.SemaphoreType.DMA((NBUF,)),
      ],
  )
  def sc_embed(x4_hbm, tab_hbm, out_hbm, idx_v, g_v, t_v, sem_i, sem_g, sem_o):
    wid = lax.axis_index("s") * NC + lax.axis_index("c")
    iota = lax.iota(jnp.int32, 16)
    zero16 = jnp.zeros((16,), jnp.int32)
    fconst = [(iota >> 3) * (8 * TP) + (iota & 7) * TP + 2 * j * (8 * TP)
              for j in range(D // 16)]

    def gather_start(c, b):
      pltpu.make_async_copy(
          tab_hbm.at[idx_v.at[c // 8, c % 8]], g_v.at[b], sem_g.at[b]).start()

    def gather_wait(c, b):
      pltpu.make_async_copy(
          tab_hbm.at[idx_v.at[c // 8, c % 8]], g_v.at[b], sem_g.at[b]).wait()

    def out_start(c, b):
      pltpu.make_async_copy(
          t_v.at[b, :, :, pl.ds(0, C)], out_hbm.at[c, :, wid],
          sem_o.at[b]).start()

    def out_wait(b):
      pltpu.make_async_copy(
          t_v.at[b, :, :, pl.ds(0, C)], out_hbm.at[0, :, wid],
          sem_o.at[b]).wait()

    def scale_transpose(b):
      @functools.partial(plsc.parallel_loop, 0, C, unroll=4)
      def body(b0):
        b0s = jnp.full((16,), b0, jnp.int32)
        for j in range(D // 16):
          vals = g_v[b, b0, pl.ds(16 * j, 16)] * SCALE
          plsc.store_scatter(t_v.at[b], [zero16, zero16, fconst[j] + b0s], vals)

    pltpu.make_async_copy(x4_hbm.at[:, wid], idx_v, sem_i).start()
    pltpu.make_async_copy(x4_hbm.at[:, wid], idx_v, sem_i).wait()

    def remap(q, carry):
      s1 = q // 8
      s0 = q % 8
      for t in range(C // 16):
        sl = pl.ds(16 * t, 16)
        v = idx_v[s1, s0, sl]
        r = v & 511
        idx_v[s1, s0, sl] = (v - r) + ((r & 255) << 1) + (r >> 8)
      return carry
    lax.fori_loop(0, NCHUNK, remap, 0, unroll=2)

    gather_start(0, 0)

    def group(g, carry):
      for b in range(NBUF):
        c = g * NBUF + b
        b1 = (b + 1) % NBUF

        @pl.when(c < NCHUNK - 1)
        def _():
          gather_start(c + 1, b1)

        gather_wait(c, b)

        @pl.when(c >= NBUF)
        def _():
          out_wait(b)

        scale_transpose(b)
        out_start(c, b)
      return carry

    lax.fori_loop(0, NGROUP, group, 0)

    for b in range(NBUF):
      out_wait(b)

  return sc_embed


_sc_embed = _build_sc_embed()


CBLK = 256
IBLK = 4096
NBULK = 999424 // IBLK
TBLK = 1024


def _make_tc_body(iblk):
  def body(t_ref, o_ref):
    for m in range(iblk // (2 * CBLK)):
      a = t_ref[:, pl.ds(2 * CBLK * m, CBLK)]
      c = t_ref[:, pl.ds(2 * CBLK * m + CBLK, CBLK)]
      o_ref[pl.ds(CBLK * m, CBLK), :] = jnp.concatenate([a, c], axis=0).T
  return body


_tc_bulk = pl.pallas_call(
    _make_tc_body(IBLK),
    grid=(NBULK,),
    in_specs=[pl.BlockSpec((D, IBLK), lambda i: (0, i))],
    out_specs=pl.BlockSpec((IBLK // 2, 128), lambda i: (i, 0)),
    out_shape=jax.ShapeDtypeStruct((500224, 128), jnp.float32),
)

_tc_tail = pl.pallas_call(
    lambda t_ref, dummy_ref, o_ref: _make_tc_body(TBLK)(t_ref, o_ref),
    grid=(1,),
    in_specs=[
        pl.BlockSpec((D, TBLK), lambda i: (0, 999424 // TBLK)),
        pl.BlockSpec((8, 128), lambda i: (0, 0)),
    ],
    out_specs=pl.BlockSpec((TBLK // 2, 128), lambda i: (999424 // TBLK, 0)),
    out_shape=jax.ShapeDtypeStruct((500224, 128), jnp.float32),
    input_output_aliases={1: 0},
)


def kernel(x, table):
  x4 = x.astype(jnp.int32).reshape(32, 128, 25, 8).transpose(2, 0, 3, 1)
  tt = table.T
  tab_lin = _tc_tail(tt, _tc_bulk(tt)).reshape(500224 * 2, D)
  w = _sc_embed(x4, tab_lin)
  return w.transpose(2, 4, 0, 1, 3).reshape(B, S, D)


# speedup vs baseline: 1.2986x; 1.2986x over previous
import functools
import math

import jax
import jax.numpy as jnp
from jax import lax
from jax.experimental import pallas as pl
from jax.experimental.pallas import tpu as pltpu
from jax.experimental.pallas import tpu_sc as plsc

D = 64
SCALE = math.sqrt(D)
B = 4096
S = 200
NC = 2
NS = 16
NW = NC * NS
C = 128
TP = C + 1
NBUF = 4
NCHUNK = S
NGROUP = NCHUNK // NBUF


def _build_sc_embed():
  mesh = plsc.VectorSubcoreMesh(core_axis_name="c", subcore_axis_name="s")

  @functools.partial(
      pl.kernel,
      mesh=mesh,
      out_type=jax.ShapeDtypeStruct((S, D // 8, B // C, 8, C), jnp.float32),
      compiler_params=pltpu.CompilerParams(use_tc_tiling_on_sc=False,
                                           needs_layout_passes=False),
      scratch_types=[
          pltpu.VMEM((S // 8, 8, C), jnp.int32),
          pltpu.VMEM((NBUF, C, D), jnp.float32),
          pltpu.VMEM((NBUF, D // 8, 8, TP), jnp.float32),
          pltpu.SemaphoreType.DMA,
          pltpu.SemaphoreType.DMA((NBUF,)),
          pltpu.SemaphoreType.DMA((NBUF,)),
      ],
  )
  def sc_embed(x4_hbm, tab_hbm, out_hbm, idx_v, g_v, t_v, sem_i, sem_g, sem_o):
    wid = lax.axis_index("s") * NC + lax.axis_index("c")
    iota = lax.iota(jnp.int32, 16)
    zero16 = jnp.zeros((16,), jnp.int32)
    fconst = [(iota >> 3) * (8 * TP) + (iota & 7) * TP + 2 * j * (8 * TP)
              for j in range(D // 16)]

    def gather_start(c, b):
      pltpu.make_async_copy(
          tab_hbm.at[idx_v.at[c // 8, c % 8]], g_v.at[b], sem_g.at[b]).start()

    def gather_wait(c, b):
      pltpu.make_async_copy(
          tab_hbm.at[idx_v.at[c // 8, c % 8]], g_v.at[b], sem_g.at[b]).wait()

    def out_start(c, b):
      pltpu.make_async_copy(
          t_v.at[b, :, :, pl.ds(0, C)], out_hbm.at[c, :, wid],
          sem_o.at[b]).start()

    def out_wait(b):
      pltpu.make_async_copy(
          t_v.at[b, :, :, pl.ds(0, C)], out_hbm.at[0, :, wid],
          sem_o.at[b]).wait()

    def scale_transpose(b):
      @functools.partial(plsc.parallel_loop, 0, C, unroll=4)
      def body(b0):
        b0s = jnp.full((16,), b0, jnp.int32)
        for j in range(D // 16):
          vals = g_v[b, b0, pl.ds(16 * j, 16)] * SCALE
          plsc.store_scatter(t_v.at[b], [zero16, zero16, fconst[j] + b0s], vals)

    pltpu.make_async_copy(x4_hbm.at[:, wid], idx_v, sem_i).start()
    pltpu.make_async_copy(x4_hbm.at[:, wid], idx_v, sem_i).wait()

    def remap(q, carry):
      s1 = q // 8
      s0 = q % 8
      for t in range(C // 16):
        sl = pl.ds(16 * t, 16)
        v = idx_v[s1, s0, sl]
        r = v & 511
        idx_v[s1, s0, sl] = (v - r) + ((r & 255) << 1) + (r >> 8)
      return carry
    lax.fori_loop(0, NCHUNK, remap, 0, unroll=2)

    gather_start(0, 0)

    def group(g, carry):
      for b in range(NBUF):
        c = g * NBUF + b
        b1 = (b + 1) % NBUF

        @pl.when(c < NCHUNK - 1)
        def _():
          gather_start(c + 1, b1)

        gather_wait(c, b)

        @pl.when(c >= NBUF)
        def _():
          out_wait(b)

        scale_transpose(b)
        out_start(c, b)
      return carry

    lax.fori_loop(0, NGROUP, group, 0)

    for b in range(NBUF):
      out_wait(b)

  return sc_embed


_sc_embed = _build_sc_embed()


CBLK = 256
IBLK = 16384
NBULK = 999424 // IBLK
TBLK = 1024


def _make_tc_body(iblk):
  def body(t_ref, o_ref):
    for m in range(iblk // (2 * CBLK)):
      a = t_ref[:, pl.ds(2 * CBLK * m, CBLK)]
      c = t_ref[:, pl.ds(2 * CBLK * m + CBLK, CBLK)]
      o_ref[pl.ds(CBLK * m, CBLK), :] = jnp.concatenate([a, c], axis=0).T
  return body


_tc_bulk = pl.pallas_call(
    _make_tc_body(IBLK),
    grid=(NBULK,),
    in_specs=[pl.BlockSpec((D, IBLK), lambda i: (0, i))],
    out_specs=pl.BlockSpec((IBLK // 2, 128), lambda i: (i, 0)),
    out_shape=jax.ShapeDtypeStruct((500224, 128), jnp.float32),
)

_tc_tail = pl.pallas_call(
    lambda t_ref, dummy_ref, o_ref: _make_tc_body(TBLK)(t_ref, o_ref),
    grid=(1,),
    in_specs=[
        pl.BlockSpec((D, TBLK), lambda i: (0, 999424 // TBLK)),
        pl.BlockSpec((8, 128), lambda i: (0, 0)),
    ],
    out_specs=pl.BlockSpec((TBLK // 2, 128), lambda i: (999424 // TBLK, 0)),
    out_shape=jax.ShapeDtypeStruct((500224, 128), jnp.float32),
    input_output_aliases={1: 0},
)


def kernel(x, table):
  x4 = x.astype(jnp.int32).reshape(32, 128, 25, 8).transpose(2, 0, 3, 1)
  tt = table.T
  tab_lin = _tc_tail(tt, _tc_bulk(tt)).reshape(500224 * 2, D)
  w = _sc_embed(x4, tab_lin)
  return w.transpose(2, 4, 0, 1, 3).reshape(B, S, D)


# speedup vs baseline: 1.3003x; 1.0013x over previous
import functools
import math

import jax
import jax.numpy as jnp
from jax import lax
from jax.experimental import pallas as pl
from jax.experimental.pallas import tpu as pltpu
from jax.experimental.pallas import tpu_sc as plsc

D = 64
SCALE = math.sqrt(D)
B = 4096
S = 200
NC = 2
NS = 16
NW = NC * NS
C = 128
TP = C + 1
NBUF = 4
NCHUNK = S
NGROUP = NCHUNK // NBUF


def _build_sc_embed():
  mesh = plsc.VectorSubcoreMesh(core_axis_name="c", subcore_axis_name="s")

  @functools.partial(
      pl.kernel,
      mesh=mesh,
      out_type=jax.ShapeDtypeStruct((S, D // 8, B // C, 8, C), jnp.float32),
      compiler_params=pltpu.CompilerParams(use_tc_tiling_on_sc=False,
                                           needs_layout_passes=False),
      scratch_types=[
          pltpu.VMEM((S // 8, 8, C), jnp.int32),
          pltpu.VMEM((NBUF, C, D), jnp.float32),
          pltpu.VMEM((NBUF, D // 8, 8, TP), jnp.float32),
          pltpu.SemaphoreType.DMA,
          pltpu.SemaphoreType.DMA((NBUF,)),
          pltpu.SemaphoreType.DMA((NBUF,)),
      ],
  )
  def sc_embed(x4_hbm, tab_hbm, out_hbm, idx_v, g_v, t_v, sem_i, sem_g, sem_o):
    wid = lax.axis_index("s") * NC + lax.axis_index("c")
    iota = lax.iota(jnp.int32, 16)
    zero16 = jnp.zeros((16,), jnp.int32)
    fconst = [(iota >> 3) * (8 * TP) + (iota & 7) * TP + 2 * j * (8 * TP)
              for j in range(D // 16)]

    def gather_start(c, b):
      pltpu.make_async_copy(
          tab_hbm.at[idx_v.at[c // 8, c % 8]], g_v.at[b], sem_g.at[b]).start()

    def gather_wait(c, b):
      pltpu.make_async_copy(
          tab_hbm.at[idx_v.at[c // 8, c % 8]], g_v.at[b], sem_g.at[b]).wait()

    def out_start(c, b):
      pltpu.make_async_copy(
          t_v.at[b, :, :, pl.ds(0, C)], out_hbm.at[c, :, wid],
          sem_o.at[b]).start()

    def out_wait(b):
      pltpu.make_async_copy(
          t_v.at[b, :, :, pl.ds(0, C)], out_hbm.at[0, :, wid],
          sem_o.at[b]).wait()

    def scale_transpose(b):
      @functools.partial(plsc.parallel_loop, 0, C, unroll=4)
      def body(b0):
        b0s = jnp.full((16,), b0, jnp.int32)
        for j in range(D // 16):
          vals = g_v[b, b0, pl.ds(16 * j, 16)] * SCALE
          plsc.store_scatter(t_v.at[b], [zero16, zero16, fconst[j] + b0s], vals)

    pltpu.make_async_copy(x4_hbm.at[:, wid], idx_v, sem_i).start()
    pltpu.make_async_copy(x4_hbm.at[:, wid], idx_v, sem_i).wait()

    def remap(q, carry):
      s1 = q // 8
      s0 = q % 8
      for t in range(C // 16):
        sl = pl.ds(16 * t, 16)
        v = idx_v[s1, s0, sl]
        r = v & 511
        idx_v[s1, s0, sl] = (v - r) + ((r & 255) << 1) + (r >> 8)
      return carry
    lax.fori_loop(0, NCHUNK, remap, 0, unroll=2)

    gather_start(0, 0)

    def group(g, carry):
      for b in range(NBUF):
        c = g * NBUF + b
        b1 = (b + 1) % NBUF

        @pl.when(c < NCHUNK - 1)
        def _():
          gather_start(c + 1, b1)

        gather_wait(c, b)

        @pl.when(c >= NBUF)
        def _():
          out_wait(b)

        scale_transpose(b)
        out_start(c, b)
      return carry

    lax.fori_loop(0, NGROUP, group, 0)

    for b in range(NBUF):
      out_wait(b)

  return sc_embed


_sc_embed = _build_sc_embed()


CBLK = 256
IBLK = 16384
NBULK = 999424 // IBLK
TBLK = 1024


def _make_tc_body(iblk):
  def body(t_ref, o_ref):
    for m in range(iblk // (2 * CBLK)):
      a = t_ref[:, pl.ds(2 * CBLK * m, CBLK)]
      c = t_ref[:, pl.ds(2 * CBLK * m + CBLK, CBLK)]
      o_ref[pl.ds(CBLK * m, CBLK), :] = jnp.concatenate([a, c], axis=0).T
  return body


_tc_bulk = pl.pallas_call(
    _make_tc_body(IBLK),
    grid=(NBULK,),
    in_specs=[pl.BlockSpec((D, IBLK), lambda i: (0, i))],
    out_specs=pl.BlockSpec((IBLK // 2, 128), lambda i: (i, 0)),
    out_shape=jax.ShapeDtypeStruct((500224, 128), jnp.float32),
    compiler_params=pltpu.CompilerParams(vmem_limit_bytes=128 * 1024 * 1024),
)

_tc_tail = pl.pallas_call(
    lambda t_ref, dummy_ref, o_ref: _make_tc_body(TBLK)(t_ref, o_ref),
    grid=(1,),
    in_specs=[
        pl.BlockSpec((D, TBLK), lambda i: (0, 999424 // TBLK)),
        pl.BlockSpec((8, 128), lambda i: (0, 0)),
    ],
    out_specs=pl.BlockSpec((TBLK // 2, 128), lambda i: (999424 // TBLK, 0)),
    out_shape=jax.ShapeDtypeStruct((500224, 128), jnp.float32),
    input_output_aliases={1: 0},
)


def kernel(x, table):
  x4 = x.astype(jnp.int32).reshape(32, 128, 25, 8).transpose(2, 0, 3, 1)
  tt = table.T
  tab_lin = _tc_tail(tt, _tc_bulk(tt)).reshape(500224 * 2, D)
  w = _sc_embed(x4, tab_lin)
  return w.transpose(2, 4, 0, 1, 3).reshape(B, S, D)
